# baseline (speedup 1.0000x reference)
import functools

import jax
import jax.numpy as jnp
from jax.experimental import pallas as pl
from jax.experimental.pallas import tpu as pltpu


def _round_up(x, m):
    return ((x + m - 1) // m) * m


def _scale_kernel(emb_ref, out_ref, *, scale):
    out_ref[...] = (emb_ref[...] * scale).astype(out_ref.dtype)


def kernel(x, embedding):
    max_seq_len, dim = embedding.shape
    seq_len = x.shape[1]
    if seq_len > max_seq_len:
        raise ValueError(f"seq_len={seq_len} exceeds max_seq_len={max_seq_len}")
    dtype = embedding.dtype
    itemsize = jnp.dtype(dtype).itemsize
    sub = max(8, 32 // itemsize)
    row_bytes = dim * itemsize

    block_rows = max(sub, _round_up(-(-seq_len // 2), sub))
    num_blocks = pl.cdiv(seq_len, block_rows)

    block_bytes = block_rows * row_bytes
    vmem_limit = 60000 * 1024

    return pl.pallas_call(
        functools.partial(_scale_kernel, scale=float(dim) ** -0.5),
        out_shape=jax.ShapeDtypeStruct((seq_len, dim), dtype),
        grid=(num_blocks,),
        in_specs=[pl.BlockSpec((block_rows, dim), lambda i: (i, 0))],
        out_specs=pl.BlockSpec((block_rows, dim), lambda i: (i, 0)),
        compiler_params=pltpu.CompilerParams(
            dimension_semantics=("parallel",),
            vmem_limit_bytes=vmem_limit,
        ),
    )(embedding)
